# initial kernel scaffold (unmeasured)
import jax
import jax.numpy as jnp
from jax import lax
from jax.experimental import pallas as pl
from jax.experimental.pallas import tpu as pltpu

N_DEV = 4


def kernel(x, w_mat):
    m_per, k = x.shape
    _, n_per = w_mat.shape
    m_total = N_DEV * m_per

    def body(x_ref, w_ref, out_ref, comm_ref, send_sems, recv_sems,
             amax_tx, amax_rx, amax_send_sems, amax_recv_sems):
        my = lax.axis_index("i")
        left = (my - 1) % N_DEV
        right = (my + 1) % N_DEV

        barrier_sem = pltpu.get_barrier_semaphore()
        for nbr in (left, right):
            pl.semaphore_signal(
                barrier_sem, inc=1,
                device_id=(nbr,), device_id_type=pl.DeviceIdType.MESH,
            )
        pl.semaphore_wait(barrier_sem, 2)

        def store_chunk(origin, chunk):
            out_ref[pl.ds(origin * m_per, m_per), :] = jnp.dot(
                chunk, w_ref[...], preferred_element_type=jnp.float32
            )

        store_chunk(my, x_ref[...])

        for h in range(N_DEV - 1):
            src = x_ref if h == 0 else comm_ref.at[h - 1]
            rdma = pltpu.make_async_remote_copy(
                src_ref=src,
                dst_ref=comm_ref.at[h],
                send_sem=send_sems.at[h],
                recv_sem=recv_sems.at[h],
                device_id=(right,),
                device_id_type=pl.DeviceIdType.MESH,
            )
            rdma.start()
            rdma.wait()
            origin = (my - h - 1) % N_DEV
            store_chunk(origin, comm_ref[h])

        my_amax = jnp.max(jnp.abs(out_ref[...]))
        amax_tx[...] = jnp.full((8, 128), my_amax, jnp.float32)
        rdmas = []
        for off in (1, 2, 3):
            tgt = (my + off) % N_DEV
            slot = 3 - off
            r = pltpu.make_async_remote_copy(
                src_ref=amax_tx,
                dst_ref=amax_rx.at[slot],
                send_sem=amax_send_sems.at[slot],
                recv_sem=amax_recv_sems.at[slot],
                device_id=(tgt,),
                device_id_type=pl.DeviceIdType.MESH,
            )
            r.start()
            rdmas.append(r)
        g_amax = my_amax
        for slot, r in enumerate(rdmas):
            r.wait_send()
            r.wait_recv()
        for slot in range(N_DEV - 1):
            g_amax = jnp.maximum(g_amax, amax_rx[slot, 0, 0])

        scale = g_amax / 448.0
        q = (out_ref[...] / scale).astype(jnp.float8_e4m3fn)
        out_ref[...] = q.astype(jnp.float32) * scale

    return pl.pallas_call(
        body,
        out_shape=jax.ShapeDtypeStruct((m_total, n_per), jnp.float32),
        in_specs=[
            pl.BlockSpec(memory_space=pltpu.VMEM),
            pl.BlockSpec(memory_space=pltpu.VMEM),
        ],
        out_specs=pl.BlockSpec(memory_space=pltpu.VMEM),
        scratch_shapes=[
            pltpu.VMEM((N_DEV - 1, m_per, k), jnp.float32),
            pltpu.SemaphoreType.DMA((N_DEV - 1,)),
            pltpu.SemaphoreType.DMA((N_DEV - 1,)),
            pltpu.VMEM((8, 128), jnp.float32),
            pltpu.VMEM((N_DEV - 1, 8, 128), jnp.float32),
            pltpu.SemaphoreType.DMA((N_DEV - 1,)),
            pltpu.SemaphoreType.DMA((N_DEV - 1,)),
        ],
        compiler_params=pltpu.CompilerParams(collective_id=0),
    )(x, w_mat)


# baseline (device time: 306169 ns/iter reference)
import jax
import jax.numpy as jnp
from jax import lax
from jax.experimental import pallas as pl
from jax.experimental.pallas import tpu as pltpu

N_DEV = 4


def kernel(x, w_mat):
    m_per, k = x.shape
    _, n_per = w_mat.shape
    m_total = N_DEV * m_per
    h_per = m_per // 2

    def body(x_hbm, w_ref, out_ref,
             cw, ccw,
             cw_send, cw_recv, ccw_send, ccw_recv, stage_sems,
             cw_credit, ccw_credit,
             amax_tx, amax_rx, amax_send_sems, amax_recv_sems):
        my = lax.axis_index("i")
        left = (my - 1) % N_DEV
        right = (my + 1) % N_DEV

        cp_cw = pltpu.make_async_copy(
            x_hbm.at[pl.ds(0, h_per)], cw.at[0], stage_sems.at[0])
        cp_ccw = pltpu.make_async_copy(
            x_hbm.at[pl.ds(h_per, h_per)], ccw.at[0], stage_sems.at[1])
        cp_cw.start()
        cp_ccw.start()
        cp_cw.wait()
        cp_ccw.wait()

        barrier_sem = pltpu.get_barrier_semaphore()
        for nbr in (left, right):
            pl.semaphore_signal(
                barrier_sem, inc=1,
                device_id=(nbr,), device_id_type=pl.DeviceIdType.MESH,
            )
        pl.semaphore_wait(barrier_sem, 2)

        def gemm(origin, half, buf):
            out_ref[pl.ds(origin * m_per + half * h_per, h_per), :] = jnp.dot(
                buf, w_ref[...], preferred_element_type=jnp.float32
            )

        def make(dirbuf, s_slot, r_slot, ssem, rsem, dev):
            return pltpu.make_async_remote_copy(
                src_ref=dirbuf.at[s_slot],
                dst_ref=dirbuf.at[r_slot],
                send_sem=ssem.at[s_slot],
                recv_sem=rsem.at[r_slot],
                device_id=(dev,),
                device_id_type=pl.DeviceIdType.MESH,
            )

        for h in range(N_DEV - 1):
            s_slot = h % 2
            r_slot = (h + 1) % 2
            if h > 0:
                pl.semaphore_wait(cw_credit, 1)
            r_cw = make(cw, s_slot, r_slot, cw_send, cw_recv, right)
            r_cw.start()
            if h > 0:
                pl.semaphore_wait(ccw_credit, 1)
            r_ccw = make(ccw, s_slot, r_slot, ccw_send, ccw_recv, left)
            r_ccw.start()

            gemm((my - h) % N_DEV, 0, cw[s_slot])
            gemm((my + h) % N_DEV, 1, ccw[s_slot])

            r_cw.wait_send()
            r_ccw.wait_send()
            if h < N_DEV - 2:
                pl.semaphore_signal(
                    cw_credit, inc=1,
                    device_id=(left,), device_id_type=pl.DeviceIdType.MESH)
                pl.semaphore_signal(
                    ccw_credit, inc=1,
                    device_id=(right,), device_id_type=pl.DeviceIdType.MESH)
            r_cw.wait_recv()
            r_ccw.wait_recv()

        last = (N_DEV - 1) % 2
        gemm((my + 1) % N_DEV, 0, cw[last])
        gemm((my - 1) % N_DEV, 1, ccw[last])

        my_amax = jnp.max(jnp.abs(out_ref[...]))
        amax_tx[...] = jnp.full((8, 128), my_amax, jnp.float32)
        rdmas = []
        for off in (1, 2, 3):
            tgt = (my + off) % N_DEV
            slot = 3 - off
            r = pltpu.make_async_remote_copy(
                src_ref=amax_tx,
                dst_ref=amax_rx.at[slot],
                send_sem=amax_send_sems.at[slot],
                recv_sem=amax_recv_sems.at[slot],
                device_id=(tgt,),
                device_id_type=pl.DeviceIdType.MESH,
            )
            r.start()
            rdmas.append(r)
        g_amax = my_amax
        for r in rdmas:
            r.wait_send()
            r.wait_recv()
        for slot in range(N_DEV - 1):
            g_amax = jnp.maximum(g_amax, amax_rx[slot, 0, 0])

        scale = g_amax / 448.0
        q = (out_ref[...] / scale).astype(jnp.float8_e4m3fn)
        out_ref[...] = q.astype(jnp.float32) * scale

    return pl.pallas_call(
        body,
        out_shape=jax.ShapeDtypeStruct((m_total, n_per), jnp.float32),
        in_specs=[
            pl.BlockSpec(memory_space=pl.ANY),
            pl.BlockSpec(memory_space=pltpu.VMEM),
        ],
        out_specs=pl.BlockSpec(memory_space=pltpu.VMEM),
        scratch_shapes=[
            pltpu.VMEM((2, h_per, k), jnp.float32),
            pltpu.VMEM((2, h_per, k), jnp.float32),
            pltpu.SemaphoreType.DMA((2,)),
            pltpu.SemaphoreType.DMA((2,)),
            pltpu.SemaphoreType.DMA((2,)),
            pltpu.SemaphoreType.DMA((2,)),
            pltpu.SemaphoreType.DMA((2,)),
            pltpu.SemaphoreType.REGULAR,
            pltpu.SemaphoreType.REGULAR,
            pltpu.VMEM((8, 128), jnp.float32),
            pltpu.VMEM((N_DEV - 1, 8, 128), jnp.float32),
            pltpu.SemaphoreType.DMA((N_DEV - 1,)),
            pltpu.SemaphoreType.DMA((N_DEV - 1,)),
        ],
        compiler_params=pltpu.CompilerParams(
            collective_id=0, vmem_limit_bytes=56 * 1024 * 1024),
    )(x, w_mat)


# device time: 211485 ns/iter; 1.4477x vs baseline; 1.4477x over previous
import jax
import jax.numpy as jnp
from jax import lax
from jax.experimental import pallas as pl
from jax.experimental.pallas import tpu as pltpu

N_DEV = 4
N_RES_SLOTS = 4


def kernel(x, w_mat):
    m_per, k = x.shape
    _, n_per = w_mat.shape
    m_total = N_DEV * m_per
    n_half = n_per // 2

    def body(x_ref, w_ref, out_ref,
             cw_w, ccw_w, res_buf,
             cw_s, cw_r, ccw_s, ccw_r, res_s, res_r,
             cw_credit, ccw_credit,
             amax_tx, amax_rx, amax_send_sems, amax_recv_sems):
        my = lax.axis_index("i")
        left = (my - 1) % N_DEV
        right = (my + 1) % N_DEV

        barrier_sem = pltpu.get_barrier_semaphore()
        for nbr in (left, right):
            pl.semaphore_signal(
                barrier_sem, inc=1,
                device_id=(nbr,), device_id_type=pl.DeviceIdType.MESH,
            )
        pl.semaphore_wait(barrier_sem, 2)

        def ring_rdma(src, buf, slot, ssem, rsem, dev):
            return pltpu.make_async_remote_copy(
                src_ref=src,
                dst_ref=buf.at[slot],
                send_sem=ssem.at[slot],
                recv_sem=rsem.at[slot],
                device_id=(dev,),
                device_id_type=pl.DeviceIdType.MESH,
            )

        res_rdmas = []

        def res_msg(buf, o, sem_slot, half):
            m = len(res_rdmas)
            slot = m % N_RES_SLOTS
            if m >= N_RES_SLOTS:
                res_rdmas[m - N_RES_SLOTS].wait_send()
            res_buf[slot, :, :] = jnp.dot(
                x_ref[...], buf[...], preferred_element_type=jnp.float32)
            r = pltpu.make_async_remote_copy(
                src_ref=res_buf.at[slot],
                dst_ref=out_ref.at[pl.ds(my * m_per, m_per),
                                   pl.ds(half * n_half, n_half)],
                send_sem=res_s.at[m],
                recv_sem=res_r.at[sem_slot],
                device_id=(o,),
                device_id_type=pl.DeviceIdType.MESH,
            )
            r.start()
            res_rdmas.append(r)

        s0 = ring_rdma(w_ref.at[:, pl.ds(0, n_half)], cw_w, 0,
                       cw_s, cw_r, right)
        t0 = ring_rdma(w_ref.at[:, pl.ds(n_half, n_half)], ccw_w, 0,
                       ccw_s, ccw_r, left)
        s0.start()
        t0.start()

        out_ref[pl.ds(my * m_per, m_per), :] = jnp.dot(
            x_ref[...], w_ref[...], preferred_element_type=jnp.float32
        )

        s0.wait_recv()
        t0.wait_recv()
        s1 = ring_rdma(cw_w.at[0], cw_w, 1, cw_s, cw_r, right)
        t1 = ring_rdma(ccw_w.at[0], ccw_w, 1, ccw_s, ccw_r, left)
        s1.start()
        t1.start()
        res_msg(cw_w.at[0], left, 0, 0)
        res_msg(ccw_w.at[0], right, 5, 1)
        s0.wait_send()
        s1.wait_send()
        pl.semaphore_signal(cw_credit, inc=1, device_id=(left,),
                            device_id_type=pl.DeviceIdType.MESH)
        t0.wait_send()
        t1.wait_send()
        pl.semaphore_signal(ccw_credit, inc=1, device_id=(right,),
                            device_id_type=pl.DeviceIdType.MESH)

        s1.wait_recv()
        t1.wait_recv()
        diag = (my + 2) % N_DEV
        pl.semaphore_wait(cw_credit, 1)
        s2 = ring_rdma(cw_w.at[1], cw_w, 0, cw_s, cw_r, right)
        s2.start()
        pl.semaphore_wait(ccw_credit, 1)
        t2 = ring_rdma(ccw_w.at[1], ccw_w, 0, ccw_s, ccw_r, left)
        t2.start()
        res_msg(cw_w.at[1], diag, 2, 0)
        res_msg(ccw_w.at[1], diag, 3, 1)

        s2.wait_recv()
        t2.wait_recv()
        res_msg(cw_w.at[0], right, 4, 0)
        res_msg(ccw_w.at[0], left, 1, 1)

        s2.wait_send()
        t2.wait_send()
        for r in res_rdmas[-N_RES_SLOTS:]:
            r.wait_send()

        for slot in range(2 * (N_DEV - 1)):
            offset = slot // 2 + 1
            half = slot % 2
            src_dev = (my + offset) % N_DEV
            recv = pltpu.make_async_remote_copy(
                src_ref=res_buf.at[slot % N_RES_SLOTS],
                dst_ref=out_ref.at[pl.ds(src_dev * m_per, m_per),
                                   pl.ds(half * n_half, n_half)],
                send_sem=res_s.at[slot],
                recv_sem=res_r.at[slot],
                device_id=(my,),
                device_id_type=pl.DeviceIdType.MESH,
            )
            recv.wait_recv()

        my_amax = jnp.max(jnp.abs(out_ref[...]))
        amax_tx[...] = jnp.full((8, 128), my_amax, jnp.float32)
        rdmas = []
        for off in (1, 2, 3):
            tgt = (my + off) % N_DEV
            slot = 3 - off
            r = pltpu.make_async_remote_copy(
                src_ref=amax_tx,
                dst_ref=amax_rx.at[slot],
                send_sem=amax_send_sems.at[slot],
                recv_sem=amax_recv_sems.at[slot],
                device_id=(tgt,),
                device_id_type=pl.DeviceIdType.MESH,
            )
            r.start()
            rdmas.append(r)
        g_amax = my_amax
        for r in rdmas:
            r.wait_send()
            r.wait_recv()
        for slot in range(N_DEV - 1):
            g_amax = jnp.maximum(g_amax, amax_rx[slot, 0, 0])

        scale = g_amax / 448.0
        q = (out_ref[...] / scale).astype(jnp.float8_e4m3fn)
        out_ref[...] = q.astype(jnp.float32) * scale

    return pl.pallas_call(
        body,
        out_shape=jax.ShapeDtypeStruct((m_total, n_per), jnp.float32),
        in_specs=[
            pl.BlockSpec(memory_space=pltpu.MemorySpace.VMEM),
            pl.BlockSpec(memory_space=pltpu.MemorySpace.VMEM),
        ],
        out_specs=pl.BlockSpec(memory_space=pltpu.MemorySpace.VMEM),
        scratch_shapes=[
            pltpu.VMEM((2, k, n_half), jnp.float32),
            pltpu.VMEM((2, k, n_half), jnp.float32),
            pltpu.VMEM((N_RES_SLOTS, m_per, n_half), jnp.float32),
            pltpu.SemaphoreType.DMA((2,)),
            pltpu.SemaphoreType.DMA((2,)),
            pltpu.SemaphoreType.DMA((2,)),
            pltpu.SemaphoreType.DMA((2,)),
            pltpu.SemaphoreType.DMA((2 * (N_DEV - 1),)),
            pltpu.SemaphoreType.DMA((2 * (N_DEV - 1),)),
            pltpu.SemaphoreType.REGULAR,
            pltpu.SemaphoreType.REGULAR,
            pltpu.VMEM((8, 128), jnp.float32),
            pltpu.VMEM((N_DEV - 1, 8, 128), jnp.float32),
            pltpu.SemaphoreType.DMA((N_DEV - 1,)),
            pltpu.SemaphoreType.DMA((N_DEV - 1,)),
        ],
        compiler_params=pltpu.CompilerParams(
            collective_id=0, vmem_limit_bytes=61 * 1024 * 1024),
    )(x, w_mat)


# device time: 207877 ns/iter; 1.4728x vs baseline; 1.0174x over previous
import jax
import jax.numpy as jnp
from jax import lax
from jax.experimental import pallas as pl
from jax.experimental.pallas import tpu as pltpu

N_DEV = 4
N_RES_SLOTS = 4


def kernel(x, w_mat):
    m_per, k = x.shape
    _, n_per = w_mat.shape
    m_total = N_DEV * m_per
    n_half = n_per // 2

    def body(x_ref, w_ref, out_ref,
             cw_w, ccw_w, res_buf,
             cw_s, cw_r, ccw_s, ccw_r, res_s, res_r,
             cw_credit, ccw_credit,
             amax_tx, amax_rx, amax_send_sems, amax_recv_sems):
        my = lax.axis_index("i")
        left = (my - 1) % N_DEV
        right = (my + 1) % N_DEV

        barrier_sem = pltpu.get_barrier_semaphore()
        for nbr in (left, right):
            pl.semaphore_signal(
                barrier_sem, inc=1,
                device_id=(nbr,), device_id_type=pl.DeviceIdType.MESH,
            )
        pl.semaphore_wait(barrier_sem, 2)

        def ring_rdma(src, buf, slot, ssem, rsem, dev):
            return pltpu.make_async_remote_copy(
                src_ref=src,
                dst_ref=buf.at[slot],
                send_sem=ssem.at[slot],
                recv_sem=rsem.at[slot],
                device_id=(dev,),
                device_id_type=pl.DeviceIdType.MESH,
            )

        res_rdmas = []
        prod_amax = []

        def res_msg(buf, o, sem_slot, half):
            m = len(res_rdmas)
            slot = m % N_RES_SLOTS
            if m >= N_RES_SLOTS:
                res_rdmas[m - N_RES_SLOTS].wait_send()
            res_buf[slot, :, :] = jnp.dot(
                x_ref[...], buf[...], preferred_element_type=jnp.float32)
            prod_amax.append(jnp.max(jnp.abs(res_buf[slot])))
            r = pltpu.make_async_remote_copy(
                src_ref=res_buf.at[slot],
                dst_ref=out_ref.at[pl.ds(my * m_per, m_per),
                                   pl.ds(half * n_half, n_half)],
                send_sem=res_s.at[m],
                recv_sem=res_r.at[sem_slot],
                device_id=(o,),
                device_id_type=pl.DeviceIdType.MESH,
            )
            r.start()
            res_rdmas.append(r)

        s0 = ring_rdma(w_ref.at[:, pl.ds(0, n_half)], cw_w, 0,
                       cw_s, cw_r, right)
        t0 = ring_rdma(w_ref.at[:, pl.ds(n_half, n_half)], ccw_w, 0,
                       ccw_s, ccw_r, left)
        s0.start()
        t0.start()

        out_ref[pl.ds(my * m_per, m_per), :] = jnp.dot(
            x_ref[...], w_ref[...], preferred_element_type=jnp.float32
        )
        prod_amax.append(
            jnp.max(jnp.abs(out_ref[pl.ds(my * m_per, m_per), :])))

        s0.wait_recv()
        t0.wait_recv()
        s1 = ring_rdma(cw_w.at[0], cw_w, 1, cw_s, cw_r, right)
        t1 = ring_rdma(ccw_w.at[0], ccw_w, 1, ccw_s, ccw_r, left)
        s1.start()
        t1.start()
        res_msg(cw_w.at[0], left, 0, 0)
        res_msg(ccw_w.at[0], right, 5, 1)
        s0.wait_send()
        s1.wait_send()
        pl.semaphore_signal(cw_credit, inc=1, device_id=(left,),
                            device_id_type=pl.DeviceIdType.MESH)
        t0.wait_send()
        t1.wait_send()
        pl.semaphore_signal(ccw_credit, inc=1, device_id=(right,),
                            device_id_type=pl.DeviceIdType.MESH)

        s1.wait_recv()
        t1.wait_recv()
        diag = (my + 2) % N_DEV
        pl.semaphore_wait(cw_credit, 1)
        s2 = ring_rdma(cw_w.at[1], cw_w, 0, cw_s, cw_r, right)
        s2.start()
        pl.semaphore_wait(ccw_credit, 1)
        t2 = ring_rdma(ccw_w.at[1], ccw_w, 0, ccw_s, ccw_r, left)
        t2.start()
        res_msg(cw_w.at[1], diag, 2, 0)
        res_msg(ccw_w.at[1], diag, 3, 1)

        s2.wait_recv()
        t2.wait_recv()
        res_msg(cw_w.at[0], right, 4, 0)
        res_msg(ccw_w.at[0], left, 1, 1)

        my_amax = prod_amax[0]
        for a in prod_amax[1:]:
            my_amax = jnp.maximum(my_amax, a)
        amax_tx[...] = jnp.full((8, 128), my_amax, jnp.float32)
        rdmas = []
        for off in (1, 2, 3):
            tgt = (my + off) % N_DEV
            slot = 3 - off
            r = pltpu.make_async_remote_copy(
                src_ref=amax_tx,
                dst_ref=amax_rx.at[slot],
                send_sem=amax_send_sems.at[slot],
                recv_sem=amax_recv_sems.at[slot],
                device_id=(tgt,),
                device_id_type=pl.DeviceIdType.MESH,
            )
            r.start()
            rdmas.append(r)
        s2.wait_send()
        t2.wait_send()
        for r in res_rdmas[-N_RES_SLOTS:]:
            r.wait_send()

        g_amax = my_amax
        for r in rdmas:
            r.wait_send()
            r.wait_recv()
        for slot in range(N_DEV - 1):
            g_amax = jnp.maximum(g_amax, amax_rx[slot, 0, 0])

        for slot in range(2 * (N_DEV - 1)):
            offset = slot // 2 + 1
            half = slot % 2
            src_dev = (my + offset) % N_DEV
            recv = pltpu.make_async_remote_copy(
                src_ref=res_buf.at[slot % N_RES_SLOTS],
                dst_ref=out_ref.at[pl.ds(src_dev * m_per, m_per),
                                   pl.ds(half * n_half, n_half)],
                send_sem=res_s.at[slot],
                recv_sem=res_r.at[slot],
                device_id=(my,),
                device_id_type=pl.DeviceIdType.MESH,
            )
            recv.wait_recv()

        scale = g_amax / 448.0
        q = (out_ref[...] / scale).astype(jnp.float8_e4m3fn)
        out_ref[...] = q.astype(jnp.float32) * scale

    return pl.pallas_call(
        body,
        out_shape=jax.ShapeDtypeStruct((m_total, n_per), jnp.float32),
        in_specs=[
            pl.BlockSpec(memory_space=pltpu.MemorySpace.VMEM),
            pl.BlockSpec(memory_space=pltpu.MemorySpace.VMEM),
        ],
        out_specs=pl.BlockSpec(memory_space=pltpu.MemorySpace.VMEM),
        scratch_shapes=[
            pltpu.VMEM((2, k, n_half), jnp.float32),
            pltpu.VMEM((2, k, n_half), jnp.float32),
            pltpu.VMEM((N_RES_SLOTS, m_per, n_half), jnp.float32),
            pltpu.SemaphoreType.DMA((2,)),
            pltpu.SemaphoreType.DMA((2,)),
            pltpu.SemaphoreType.DMA((2,)),
            pltpu.SemaphoreType.DMA((2,)),
            pltpu.SemaphoreType.DMA((2 * (N_DEV - 1),)),
            pltpu.SemaphoreType.DMA((2 * (N_DEV - 1),)),
            pltpu.SemaphoreType.REGULAR,
            pltpu.SemaphoreType.REGULAR,
            pltpu.VMEM((8, 128), jnp.float32),
            pltpu.VMEM((N_DEV - 1, 8, 128), jnp.float32),
            pltpu.SemaphoreType.DMA((N_DEV - 1,)),
            pltpu.SemaphoreType.DMA((N_DEV - 1,)),
        ],
        compiler_params=pltpu.CompilerParams(
            collective_id=0, vmem_limit_bytes=61 * 1024 * 1024),
    )(x, w_mat)


# device time: 206298 ns/iter; 1.4841x vs baseline; 1.0077x over previous
import jax
import jax.numpy as jnp
from jax import lax
from jax.experimental import pallas as pl
from jax.experimental.pallas import tpu as pltpu

N_DEV = 4
N_RES_SLOTS = 4


def kernel(x, w_mat):
    m_per, k = x.shape
    _, n_per = w_mat.shape
    m_total = N_DEV * m_per
    n_half = n_per // 2

    def body(x_ref, w_ref, out_ref,
             cw_w, ccw_w, res_buf,
             cw_s, cw_r, ccw_s, ccw_r, res_s, res_r,
             cw_credit, ccw_credit,
             amax_tx, amax_rx, amax_send_sems, amax_recv_sems):
        my = lax.axis_index("i")
        left = (my - 1) % N_DEV
        right = (my + 1) % N_DEV

        barrier_sem = pltpu.get_barrier_semaphore()
        for nbr in (left, right):
            pl.semaphore_signal(
                barrier_sem, inc=1,
                device_id=(nbr,), device_id_type=pl.DeviceIdType.MESH,
            )
        pl.semaphore_wait(barrier_sem, 2)

        def ring_rdma(src, buf, slot, ssem, rsem, dev):
            return pltpu.make_async_remote_copy(
                src_ref=src,
                dst_ref=buf.at[slot],
                send_sem=ssem.at[slot],
                recv_sem=rsem.at[slot],
                device_id=(dev,),
                device_id_type=pl.DeviceIdType.MESH,
            )

        res_rdmas = []
        prod_amax = []

        def res_msg(buf, o, sem_slot, half):
            m = len(res_rdmas)
            slot = m % N_RES_SLOTS
            if m >= N_RES_SLOTS:
                res_rdmas[m - N_RES_SLOTS].wait_send()
            res_buf[slot, :, :] = jnp.dot(
                x_ref[...], buf[...], preferred_element_type=jnp.float32)
            prod_amax.append(jnp.max(jnp.abs(res_buf[slot])))
            r = pltpu.make_async_remote_copy(
                src_ref=res_buf.at[slot],
                dst_ref=out_ref.at[pl.ds(my * m_per, m_per),
                                   pl.ds(half * n_half, n_half)],
                send_sem=res_s.at[m],
                recv_sem=res_r.at[sem_slot],
                device_id=(o,),
                device_id_type=pl.DeviceIdType.MESH,
            )
            r.start()
            res_rdmas.append(r)

        s0 = ring_rdma(w_ref.at[:, pl.ds(0, n_half)], cw_w, 0,
                       cw_s, cw_r, right)
        t0 = ring_rdma(w_ref.at[:, pl.ds(n_half, n_half)], ccw_w, 0,
                       ccw_s, ccw_r, left)
        s0.start()
        t0.start()

        out_ref[pl.ds(my * m_per, m_per), :] = jnp.dot(
            x_ref[...], w_ref[...], preferred_element_type=jnp.float32
        )
        prod_amax.append(
            jnp.max(jnp.abs(out_ref[pl.ds(my * m_per, m_per), :])))

        s0.wait_recv()
        t0.wait_recv()
        s1 = ring_rdma(cw_w.at[0], cw_w, 1, cw_s, cw_r, right)
        t1 = ring_rdma(ccw_w.at[0], ccw_w, 1, ccw_s, ccw_r, left)
        s1.start()
        t1.start()
        res_msg(cw_w.at[0], left, 0, 0)
        res_msg(ccw_w.at[0], right, 5, 1)
        s0.wait_send()
        s1.wait_send()
        pl.semaphore_signal(cw_credit, inc=1, device_id=(left,),
                            device_id_type=pl.DeviceIdType.MESH)
        t0.wait_send()
        t1.wait_send()
        pl.semaphore_signal(ccw_credit, inc=1, device_id=(right,),
                            device_id_type=pl.DeviceIdType.MESH)

        s1.wait_recv()
        t1.wait_recv()
        diag = (my + 2) % N_DEV
        n_q = n_half // 2
        pl.semaphore_wait(cw_credit, 1)
        s2a = pltpu.make_async_remote_copy(
            src_ref=cw_w.at[1, :, pl.ds(0, n_q)],
            dst_ref=cw_w.at[0, :, pl.ds(0, n_q)],
            send_sem=cw_s.at[0], recv_sem=cw_r.at[0],
            device_id=(right,), device_id_type=pl.DeviceIdType.MESH)
        s2b = pltpu.make_async_remote_copy(
            src_ref=cw_w.at[1, :, pl.ds(n_q, n_q)],
            dst_ref=cw_w.at[0, :, pl.ds(n_q, n_q)],
            send_sem=cw_s.at[1], recv_sem=cw_r.at[1],
            device_id=(right,), device_id_type=pl.DeviceIdType.MESH)
        s2a.start()
        s2b.start()
        pl.semaphore_wait(ccw_credit, 1)
        t2a = pltpu.make_async_remote_copy(
            src_ref=ccw_w.at[1, :, pl.ds(0, n_q)],
            dst_ref=ccw_w.at[0, :, pl.ds(0, n_q)],
            send_sem=ccw_s.at[0], recv_sem=ccw_r.at[0],
            device_id=(left,), device_id_type=pl.DeviceIdType.MESH)
        t2b = pltpu.make_async_remote_copy(
            src_ref=ccw_w.at[1, :, pl.ds(n_q, n_q)],
            dst_ref=ccw_w.at[0, :, pl.ds(n_q, n_q)],
            send_sem=ccw_s.at[1], recv_sem=ccw_r.at[1],
            device_id=(left,), device_id_type=pl.DeviceIdType.MESH)
        t2a.start()
        t2b.start()
        res_msg(cw_w.at[1], diag, 2, 0)
        res_msg(ccw_w.at[1], diag, 3, 1)

        s2a.wait_recv()
        res_rdmas[0].wait_send()
        res_buf[0, :, pl.ds(0, n_q)] = jnp.dot(
            x_ref[...], cw_w[0, :, pl.ds(0, n_q)],
            preferred_element_type=jnp.float32)
        t2a.wait_recv()
        res_rdmas[1].wait_send()
        res_buf[1, :, pl.ds(0, n_q)] = jnp.dot(
            x_ref[...], ccw_w[0, :, pl.ds(0, n_q)],
            preferred_element_type=jnp.float32)
        s2b.wait_recv()
        res_buf[0, :, pl.ds(n_q, n_q)] = jnp.dot(
            x_ref[...], cw_w[0, :, pl.ds(n_q, n_q)],
            preferred_element_type=jnp.float32)
        prod_amax.append(jnp.max(jnp.abs(res_buf[0])))
        m4 = pltpu.make_async_remote_copy(
            src_ref=res_buf.at[0],
            dst_ref=out_ref.at[pl.ds(my * m_per, m_per), pl.ds(0, n_half)],
            send_sem=res_s.at[4], recv_sem=res_r.at[4],
            device_id=(right,), device_id_type=pl.DeviceIdType.MESH)
        m4.start()
        res_rdmas.append(m4)
        t2b.wait_recv()
        res_buf[1, :, pl.ds(n_q, n_q)] = jnp.dot(
            x_ref[...], ccw_w[0, :, pl.ds(n_q, n_q)],
            preferred_element_type=jnp.float32)
        prod_amax.append(jnp.max(jnp.abs(res_buf[1])))
        m5 = pltpu.make_async_remote_copy(
            src_ref=res_buf.at[1],
            dst_ref=out_ref.at[pl.ds(my * m_per, m_per),
                               pl.ds(n_half, n_half)],
            send_sem=res_s.at[5], recv_sem=res_r.at[1],
            device_id=(left,), device_id_type=pl.DeviceIdType.MESH)
        m5.start()
        res_rdmas.append(m5)

        my_amax = prod_amax[0]
        for a in prod_amax[1:]:
            my_amax = jnp.maximum(my_amax, a)
        amax_tx[...] = jnp.full((8, 128), my_amax, jnp.float32)
        rdmas = []
        for off in (1, 2, 3):
            tgt = (my + off) % N_DEV
            slot = 3 - off
            r = pltpu.make_async_remote_copy(
                src_ref=amax_tx,
                dst_ref=amax_rx.at[slot],
                send_sem=amax_send_sems.at[slot],
                recv_sem=amax_recv_sems.at[slot],
                device_id=(tgt,),
                device_id_type=pl.DeviceIdType.MESH,
            )
            r.start()
            rdmas.append(r)
        for r in (s2a, s2b, t2a, t2b):
            r.wait_send()
        for r in res_rdmas[2:]:
            r.wait_send()

        g_amax = my_amax
        for r in rdmas:
            r.wait_send()
            r.wait_recv()
        for slot in range(N_DEV - 1):
            g_amax = jnp.maximum(g_amax, amax_rx[slot, 0, 0])

        for slot in range(2 * (N_DEV - 1)):
            offset = slot // 2 + 1
            half = slot % 2
            src_dev = (my + offset) % N_DEV
            recv = pltpu.make_async_remote_copy(
                src_ref=res_buf.at[slot % N_RES_SLOTS],
                dst_ref=out_ref.at[pl.ds(src_dev * m_per, m_per),
                                   pl.ds(half * n_half, n_half)],
                send_sem=res_s.at[slot],
                recv_sem=res_r.at[slot],
                device_id=(my,),
                device_id_type=pl.DeviceIdType.MESH,
            )
            recv.wait_recv()

        scale = g_amax / 448.0
        q = (out_ref[...] / scale).astype(jnp.float8_e4m3fn)
        out_ref[...] = q.astype(jnp.float32) * scale

    return pl.pallas_call(
        body,
        out_shape=jax.ShapeDtypeStruct((m_total, n_per), jnp.float32),
        in_specs=[
            pl.BlockSpec(memory_space=pltpu.MemorySpace.VMEM),
            pl.BlockSpec(memory_space=pltpu.MemorySpace.VMEM),
        ],
        out_specs=pl.BlockSpec(memory_space=pltpu.MemorySpace.VMEM),
        scratch_shapes=[
            pltpu.VMEM((2, k, n_half), jnp.float32),
            pltpu.VMEM((2, k, n_half), jnp.float32),
            pltpu.VMEM((N_RES_SLOTS, m_per, n_half), jnp.float32),
            pltpu.SemaphoreType.DMA((2,)),
            pltpu.SemaphoreType.DMA((2,)),
            pltpu.SemaphoreType.DMA((2,)),
            pltpu.SemaphoreType.DMA((2,)),
            pltpu.SemaphoreType.DMA((2 * (N_DEV - 1),)),
            pltpu.SemaphoreType.DMA((2 * (N_DEV - 1),)),
            pltpu.SemaphoreType.REGULAR,
            pltpu.SemaphoreType.REGULAR,
            pltpu.VMEM((8, 128), jnp.float32),
            pltpu.VMEM((N_DEV - 1, 8, 128), jnp.float32),
            pltpu.SemaphoreType.DMA((N_DEV - 1,)),
            pltpu.SemaphoreType.DMA((N_DEV - 1,)),
        ],
        compiler_params=pltpu.CompilerParams(
            collective_id=0, vmem_limit_bytes=61 * 1024 * 1024),
    )(x, w_mat)
